# Initial kernel scaffold; baseline (speedup 1.0000x reference)
import jax
import jax.numpy as jnp
from jax.experimental import pallas as pl


def kernel(x):
    raise NotImplementedError("write your pallas kernel here")


# speedup vs baseline: 1.4812x; 1.4812x over previous
import jax
import jax.numpy as jnp
from jax import lax
from jax.experimental import pallas as pl


def _body(x_ref, o_ref):
    xb = x_ref[0]
    C = xb.shape[0]
    m = jnp.max(xb, axis=0)
    iota = lax.broadcasted_iota(jnp.int32, xb.shape, 0)
    idx = jnp.min(jnp.where(xb == m[None], iota, C), axis=0)
    contrib = jnp.sum(jnp.where(iota == idx[None], xb, 0.0), axis=(1, 2))

    @pl.when(pl.program_id(1) == 0)
    def _():
        o_ref[...] = jnp.zeros_like(o_ref)

    o_ref[0, 0, :] += contrib


def kernel(x):
    B, C, H, W = x.shape
    HC = 64
    out = pl.pallas_call(
        _body,
        grid=(B, H // HC),
        in_specs=[pl.BlockSpec((1, C, HC, W), lambda b, h: (b, 0, h, 0))],
        out_specs=pl.BlockSpec((1, 1, C), lambda b, h: (b, 0, 0)),
        out_shape=jax.ShapeDtypeStruct((B, 1, C), x.dtype),
    )(x)
    return out.reshape(B, C)
